# Initial kernel scaffold; baseline (speedup 1.0000x reference)
import jax
import jax.numpy as jnp
from jax.experimental import pallas as pl


def kernel(input_, src_lengths, weight):
    raise NotImplementedError("write your pallas kernel here")


# speedup vs baseline: 9.0813x; 9.0813x over previous
import functools

import jax
import jax.numpy as jnp
from jax import lax
from jax.experimental import pallas as pl
from jax.experimental.pallas import tpu as pltpu
from jax.experimental.pallas import tpu_sc as plsc

_NC = 2
_NS = 16
_NW = _NC * _NS
_LANES = 16


@functools.lru_cache(maxsize=None)
def _make_kernel(B, L, V, D, CB):
    del V
    BPW = B // _NW
    NCH = BPW // CB
    RPC = CB * L

    mesh = plsc.VectorSubcoreMesh(
        core_axis_name="c", subcore_axis_name="s",
        num_cores=_NC, num_subcores=_NS)

    @functools.partial(
        pl.kernel,
        out_type=jax.ShapeDtypeStruct((B, D), jnp.float32),
        mesh=mesh,
        scratch_types=[
            pltpu.VMEM((RPC,), jnp.int32),
            pltpu.VMEM((RPC,), jnp.int32),
            pltpu.VMEM((RPC, D), jnp.float32),
            pltpu.VMEM((RPC, D), jnp.float32),
            pltpu.VMEM((BPW,), jnp.int32),
            pltpu.VMEM((BPW,), jnp.float32),
            pltpu.VMEM((BPW, D), jnp.float32),
            pltpu.SemaphoreType.DMA,
            pltpu.SemaphoreType.DMA,
            pltpu.SemaphoreType.DMA,
            pltpu.SemaphoreType.DMA,
        ],
        compiler_params=pltpu.CompilerParams(use_tc_tiling_on_sc=False),
    )
    def emb_kernel(idx_hbm, len_hbm, w_hbm, out_hbm,
                   idx0, idx1, rows0, rows1, lens_v, inv_v, out_v,
                   sem_i0, sem_i1, sem_r0, sem_r1):
        wid = lax.axis_index("s") * _NC + lax.axis_index("c")
        base = wid * BPW

        pltpu.sync_copy(len_hbm.at[pl.ds(base, BPW)], lens_v)

        def inv_body(g, carry):
            lv = lens_v[pl.ds(g * _LANES, _LANES)]
            inv_v[pl.ds(g * _LANES, _LANES)] = 1.0 / lv.astype(jnp.float32)
            return carry

        lax.fori_loop(0, BPW // _LANES, inv_body, 0)

        idx_bufs = (idx0, idx1)
        row_bufs = (rows0, rows1)
        idx_sems = (sem_i0, sem_i1)
        row_sems = (sem_r0, sem_r1)

        def start_idx(c, p):
            pltpu.async_copy(
                idx_hbm.at[pl.ds((base + c * CB) * L, RPC)],
                idx_bufs[p], idx_sems[p])

        def wait_idx(p):
            pltpu.make_async_copy(
                idx_hbm.at[pl.ds(0, RPC)], idx_bufs[p], idx_sems[p]).wait()

        def start_gather(p):
            pltpu.async_copy(w_hbm.at[idx_bufs[p]], row_bufs[p], row_sems[p])

        def wait_gather(p):
            pltpu.make_async_copy(
                w_hbm.at[idx_bufs[p]], row_bufs[p], row_sems[p]).wait()

        def process(c, p):
            wait_gather(p)

            @pl.when(c + 2 < NCH)
            def _():
                start_idx(c + 2, p)

            @pl.when(c + 1 < NCH)
            def _():
                wait_idx(1 - p)
                start_gather(1 - p)

            rows = row_bufs[p]
            for b in range(CB):
                rbase = b * L

                def jbody(j, accs):
                    a0, a1, a2, a3 = accs
                    r = rbase + j * 4
                    a0 = a0 + rows[r, :]
                    a1 = a1 + rows[r + 1, :]
                    a2 = a2 + rows[r + 2, :]
                    a3 = a3 + rows[r + 3, :]
                    return (a0, a1, a2, a3)

                z = jnp.zeros((_LANES,), jnp.float32)
                a0, a1, a2, a3 = lax.fori_loop(0, L // 4, jbody, (z, z, z, z))
                acc = (a0 + a1) + (a2 + a3)
                bb = c * CB + b
                iv = inv_v[pl.ds((c - p) * CB, _LANES)]
                lane = jnp.full((_LANES, 1), p * CB + b, jnp.int32)
                cvec = lax.gather(
                    iv, lane,
                    dimension_numbers=lax.GatherDimensionNumbers(
                        offset_dims=(), collapsed_slice_dims=(0,),
                        start_index_map=(0,)),
                    slice_sizes=(1,),
                    mode=lax.GatherScatterMode.PROMISE_IN_BOUNDS)
                out_v[bb, :] = acc * cvec

        start_idx(0, 0)
        start_idx(1, 1)
        wait_idx(0)
        start_gather(0)

        def chunk_pair(g, carry):
            process(2 * g, 0)
            process(2 * g + 1, 1)
            return carry

        lax.fori_loop(0, NCH // 2, chunk_pair, 0)

        pltpu.sync_copy(out_v, out_hbm.at[pl.ds(base, BPW)])

    return emb_kernel


def kernel(input_, src_lengths, weight):
    B, L = input_.shape
    V, D = weight.shape
    k = _make_kernel(B, L, V, D, CB=8)
    return k(input_.reshape(B * L), src_lengths, weight)


# speedup vs baseline: 9.0862x; 1.0005x over previous
import functools

import jax
import jax.numpy as jnp
from jax import lax
from jax.experimental import pallas as pl
from jax.experimental.pallas import tpu as pltpu
from jax.experimental.pallas import tpu_sc as plsc

_NC = 2
_NS = 16
_NW = _NC * _NS
_LANES = 16


@functools.lru_cache(maxsize=None)
def _make_kernel(B, L, V, D, CB, SPLIT):
    del V
    BPW = B // _NW
    NCH = BPW // CB
    RPC = CB * L
    RPS = RPC // SPLIT

    mesh = plsc.VectorSubcoreMesh(
        core_axis_name="c", subcore_axis_name="s",
        num_cores=_NC, num_subcores=_NS)

    @functools.partial(
        pl.kernel,
        out_type=jax.ShapeDtypeStruct((B, D), jnp.float32),
        mesh=mesh,
        scratch_types=[
            pltpu.VMEM((RPC,), jnp.int32),
            pltpu.VMEM((RPC,), jnp.int32),
            pltpu.VMEM((RPC, D), jnp.float32),
            pltpu.VMEM((RPC, D), jnp.float32),
            pltpu.VMEM((BPW,), jnp.int32),
            pltpu.VMEM((BPW,), jnp.float32),
            pltpu.VMEM((BPW, D), jnp.float32),
            pltpu.SemaphoreType.DMA,
            pltpu.SemaphoreType.DMA,
            pltpu.SemaphoreType.DMA,
            pltpu.SemaphoreType.DMA,
        ],
        compiler_params=pltpu.CompilerParams(use_tc_tiling_on_sc=False),
    )
    def emb_kernel(idx_hbm, len_hbm, w_hbm, out_hbm,
                   idx0, idx1, rows0, rows1, lens_v, inv_v, out_v,
                   sem_i0, sem_i1, sem_r0, sem_r1):
        wid = lax.axis_index("s") * _NC + lax.axis_index("c")
        base = wid * BPW

        pltpu.sync_copy(len_hbm.at[pl.ds(base, BPW)], lens_v)

        def inv_body(g, carry):
            lv = lens_v[pl.ds(g * _LANES, _LANES)]
            inv_v[pl.ds(g * _LANES, _LANES)] = 1.0 / lv.astype(jnp.float32)
            return carry

        lax.fori_loop(0, BPW // _LANES, inv_body, 0)

        idx_bufs = (idx0, idx1)
        row_bufs = (rows0, rows1)
        idx_sems = (sem_i0, sem_i1)
        row_sems = (sem_r0, sem_r1)

        def start_idx(c, p):
            pltpu.async_copy(
                idx_hbm.at[pl.ds((base + c * CB) * L, RPC)],
                idx_bufs[p], idx_sems[p])

        def wait_idx(p):
            pltpu.make_async_copy(
                idx_hbm.at[pl.ds(0, RPC)], idx_bufs[p], idx_sems[p]).wait()

        def start_gather(p):
            for s in range(SPLIT):
                pltpu.async_copy(
                    w_hbm.at[idx_bufs[p].at[pl.ds(s * RPS, RPS)]],
                    row_bufs[p].at[pl.ds(s * RPS, RPS), :],
                    row_sems[p])

        def wait_gather(p):
            for s in range(SPLIT):
                pltpu.make_async_copy(
                    w_hbm.at[idx_bufs[p].at[pl.ds(s * RPS, RPS)]],
                    row_bufs[p].at[pl.ds(s * RPS, RPS), :],
                    row_sems[p]).wait()

        def process(c, p):
            wait_gather(p)

            @pl.when(c + 2 < NCH)
            def _():
                start_idx(c + 2, p)

            @pl.when(c + 1 < NCH)
            def _():
                wait_idx(1 - p)
                start_gather(1 - p)

            rows = row_bufs[p]
            for b in range(CB):
                rbase = b * L

                def jbody(j, accs):
                    a0, a1, a2, a3 = accs
                    r = rbase + j * 4
                    a0 = a0 + rows[r, :]
                    a1 = a1 + rows[r + 1, :]
                    a2 = a2 + rows[r + 2, :]
                    a3 = a3 + rows[r + 3, :]
                    return (a0, a1, a2, a3)

                z = jnp.zeros((_LANES,), jnp.float32)
                a0, a1, a2, a3 = lax.fori_loop(0, L // 4, jbody, (z, z, z, z))
                acc = (a0 + a1) + (a2 + a3)
                bb = c * CB + b
                iv = inv_v[pl.ds((c - p) * CB, _LANES)]
                lane = jnp.full((_LANES, 1), p * CB + b, jnp.int32)
                cvec = lax.gather(
                    iv, lane,
                    dimension_numbers=lax.GatherDimensionNumbers(
                        offset_dims=(), collapsed_slice_dims=(0,),
                        start_index_map=(0,)),
                    slice_sizes=(1,),
                    mode=lax.GatherScatterMode.PROMISE_IN_BOUNDS)
                out_v[bb, :] = acc * cvec

        start_idx(0, 0)
        start_idx(1, 1)
        wait_idx(0)
        start_gather(0)

        def chunk_pair(g, carry):
            process(2 * g, 0)
            process(2 * g + 1, 1)
            return carry

        lax.fori_loop(0, NCH // 2, chunk_pair, 0)

        pltpu.sync_copy(out_v, out_hbm.at[pl.ds(base, BPW)])

    return emb_kernel


def kernel(input_, src_lengths, weight):
    B, L = input_.shape
    V, D = weight.shape
    k = _make_kernel(B, L, V, D, CB=8, SPLIT=4)
    return k(input_.reshape(B * L), src_lengths, weight)


# speedup vs baseline: 9.2140x; 1.0141x over previous
import functools

import jax
import jax.numpy as jnp
from jax import lax
from jax.experimental import pallas as pl
from jax.experimental.pallas import tpu as pltpu
from jax.experimental.pallas import tpu_sc as plsc

_NC = 2
_NS = 16
_NW = _NC * _NS
_LANES = 16


@functools.lru_cache(maxsize=None)
def _make_kernel(B, L, V, D, CB):
    del V
    BPW = B // _NW
    NCH = BPW // CB
    RPC = CB * L
    assert CB == _LANES and NCH % 2 == 0

    mesh = plsc.VectorSubcoreMesh(
        core_axis_name="c", subcore_axis_name="s",
        num_cores=_NC, num_subcores=_NS)

    @functools.partial(
        pl.kernel,
        out_type=jax.ShapeDtypeStruct((B, D), jnp.float32),
        mesh=mesh,
        scratch_types=[
            pltpu.VMEM((L, CB), jnp.int32),
            pltpu.VMEM((L, CB), jnp.int32),
            pltpu.VMEM((RPC,), jnp.int32),
            pltpu.VMEM((RPC,), jnp.int32),
            pltpu.VMEM((RPC, D), jnp.float32),
            pltpu.VMEM((RPC, D), jnp.float32),
            pltpu.VMEM((BPW,), jnp.int32),
            pltpu.VMEM((BPW,), jnp.float32),
            pltpu.VMEM((BPW, D), jnp.float32),
            pltpu.SemaphoreType.DMA,
            pltpu.SemaphoreType.DMA,
            pltpu.SemaphoreType.DMA,
            pltpu.SemaphoreType.DMA,
        ],
        compiler_params=pltpu.CompilerParams(use_tc_tiling_on_sc=False),
    )
    def emb_kernel(idxT_hbm, len_hbm, w_hbm, out_hbm,
                   idx0, idx1, fidx0, fidx1, rows0, rows1,
                   lens_v, inv_v, out_v,
                   sem_i0, sem_i1, sem_r0, sem_r1):
        wid = lax.axis_index("s") * _NC + lax.axis_index("c")
        base = wid * BPW

        pltpu.sync_copy(len_hbm.at[pl.ds(base, BPW)], lens_v)

        def inv_body(g, carry):
            lv = lens_v[pl.ds(g * _LANES, _LANES)]
            inv_v[pl.ds(g * _LANES, _LANES)] = 1.0 / lv.astype(jnp.float32)
            return carry

        lax.fori_loop(0, BPW // _LANES, inv_body, 0)

        idx_bufs = (idx0, idx1)
        fidx_bufs = (fidx0, fidx1)
        row_bufs = (rows0, rows1)
        idx_sems = (sem_i0, sem_i1)
        row_sems = (sem_r0, sem_r1)

        def start_idx(c, p):
            pltpu.async_copy(
                idxT_hbm.at[:, pl.ds(base + c * CB, CB)],
                idx_bufs[p], idx_sems[p])

        def wait_idx(p):
            pltpu.make_async_copy(
                idxT_hbm.at[:, pl.ds(0, CB)],
                idx_bufs[p], idx_sems[p]).wait()

        def flatten_idx(p):
            src, dst = idx_bufs[p], fidx_bufs[p]

            def fbody(l, carry):
                dst[pl.ds(l * CB, CB)] = src[l, :]
                return carry

            lax.fori_loop(0, L, fbody, 0)

        def start_gather(p):
            pltpu.async_copy(
                w_hbm.at[fidx_bufs[p]], row_bufs[p], row_sems[p])

        def wait_gather(p):
            pltpu.make_async_copy(
                w_hbm.at[fidx_bufs[p]], row_bufs[p], row_sems[p]).wait()

        def process(c, p):
            wait_gather(p)

            @pl.when(c + 2 < NCH)
            def _():
                start_idx(c + 2, p)

            @pl.when(c + 1 < NCH)
            def _():
                wait_idx(1 - p)
                flatten_idx(1 - p)
                start_gather(1 - p)

            rows = row_bufs[p]
            for b in range(CB):
                def jbody(j, accs):
                    a0, a1, a2, a3 = accs
                    r = (j * 4) * CB + b
                    a0 = a0 + rows[r, :]
                    a1 = a1 + rows[r + CB, :]
                    a2 = a2 + rows[r + 2 * CB, :]
                    a3 = a3 + rows[r + 3 * CB, :]
                    return (a0, a1, a2, a3)

                z = jnp.zeros((_LANES,), jnp.float32)
                a0, a1, a2, a3 = lax.fori_loop(0, L // 4, jbody, (z, z, z, z))
                acc = (a0 + a1) + (a2 + a3)
                bb = c * CB + b
                iv = inv_v[pl.ds(c * CB, _LANES)]
                lane = jnp.full((_LANES, 1), b, jnp.int32)
                cvec = lax.gather(
                    iv, lane,
                    dimension_numbers=lax.GatherDimensionNumbers(
                        offset_dims=(), collapsed_slice_dims=(0,),
                        start_index_map=(0,)),
                    slice_sizes=(1,),
                    mode=lax.GatherScatterMode.PROMISE_IN_BOUNDS)
                out_v[bb, :] = acc * cvec

        start_idx(0, 0)
        start_idx(1, 1)
        wait_idx(0)
        flatten_idx(0)
        start_gather(0)

        def chunk_pair(g, carry):
            process(2 * g, 0)
            process(2 * g + 1, 1)
            return carry

        lax.fori_loop(0, NCH // 2, chunk_pair, 0)

        pltpu.sync_copy(out_v, out_hbm.at[pl.ds(base, BPW)])

    return emb_kernel


def kernel(input_, src_lengths, weight):
    B, L = input_.shape
    V, D = weight.shape
    k = _make_kernel(B, L, V, D, CB=16)
    return k(input_.T, src_lengths, weight)
